# Initial kernel scaffold; baseline (speedup 1.0000x reference)
import jax
import jax.numpy as jnp
from jax.experimental import pallas as pl


def kernel(x, edge_index, edge_w, W, b, prelu_a):
    raise NotImplementedError("write your pallas kernel here")


# speedup vs baseline: 2.6908x; 2.6908x over previous
import functools

import jax
import jax.numpy as jnp
from jax import lax
from jax.experimental import pallas as pl
from jax.experimental.pallas import tpu as pltpu
from jax.experimental.pallas import tpu_sc as plsc

N = 10000
B = 100
NPER = 100
D = 256
H = 128
E = 160000
NC = 2
NS = 16
K = 128
G = 8
NG = 10
CH = G * NG
EPT = CH * K
EP = NS * EPT
RB = 2000


def _mm_body(x_ref, w_ref, b_ref, anch_ref, feat_ref, aout_ref):
    i = pl.program_id(0)
    xb = x_ref[...]
    row = lax.broadcasted_iota(jnp.int32, (RB, 1), 0) + i * RB
    xb = jnp.where(row % NPER == 0, 0.0, xb)
    f = jnp.dot(xb, w_ref[...], preferred_element_type=jnp.float32)
    feat_ref[0] = f[:, :H]
    feat_ref[1] = f[:, H:]

    @pl.when(i == 0)
    def _():
        aout_ref[...] = (
            jnp.dot(anch_ref[...], w_ref[...], preferred_element_type=jnp.float32)
            + b_ref[...]
        )


def _phase_a(x, W, b2, anchors):
    return pl.pallas_call(
        _mm_body,
        grid=(N // RB,),
        in_specs=[
            pl.BlockSpec((RB, D), lambda i: (i, 0)),
            pl.BlockSpec((D, D), lambda i: (0, 0)),
            pl.BlockSpec((1, D), lambda i: (0, 0)),
            pl.BlockSpec((B, D), lambda i: (0, 0)),
        ],
        out_specs=[
            pl.BlockSpec((2, RB, H), lambda i: (0, i, 0)),
            pl.BlockSpec((B, D), lambda i: (0, 0)),
        ],
        out_shape=[
            jax.ShapeDtypeStruct((2, N, H), jnp.float32),
            jax.ShapeDtypeStruct((B, D), jnp.float32),
        ],
    )(x, W, b2, anchors)


def _sc_body(feat_hbm, src_hbm, dst_hbm, w_hbm, b_hbm, a_hbm, out_hbm,
             eg_src, eg_dst, eg_w, buf, gbuf, b_v, a_v, prow, sh_h):
    half = lax.axis_index("c")
    t = lax.axis_index("s")

    def _zrow(r, _):
        for c in range(H // 16):
            buf[r, pl.ds(c * 16, 16)] = jnp.zeros((16,), jnp.float32)
        return 0

    lax.fori_loop(0, K, _zrow, 0)
    ZR = 125
    for i in range(5):
        pltpu.sync_copy(buf.at[pl.ds(0, ZR)],
                        sh_h.at[pl.ds(t * 625 + i * ZR, ZR)])

    pltpu.sync_copy(b_hbm.at[half], b_v)
    pltpu.sync_copy(a_hbm, a_v)

    off = half * N

    plsc.subcore_barrier()

    def _group(gi, _):
        gsl = pl.ds(gi * G, G)
        pltpu.sync_copy(src_hbm.at[t, gsl], eg_src)
        pltpu.sync_copy(dst_hbm.at[t, gsl], eg_dst)
        pltpu.sync_copy(w_hbm.at[t, gsl], eg_w)
        for j in range(G):
            for c in range(K // 16):
                sl = pl.ds(c * 16, 16)
                eg_src[j, sl] = eg_src[j, sl] + off
        for j in range(G):
            pltpu.sync_copy(feat_hbm.at[eg_src.at[j]], buf)

            def _scale(k, _):
                wk = plsc.load_gather(
                    eg_w, [jnp.full((16,), j, jnp.int32),
                           jnp.full((16,), k, jnp.int32)])
                for c in range(H // 16):
                    sl = pl.ds(c * 16, 16)
                    buf[k, sl] = buf[k, sl] * wk
                return 0

            lax.fori_loop(0, K, _scale, 0)
            pltpu.sync_copy(buf, sh_h.at[eg_dst.at[j]], add=True)
        return 0

    lax.fori_loop(0, NG, _group, 0)

    plsc.subcore_barrier()

    alpha = a_v[...]
    for gi in range(7):
        g = t + NS * gi

        @pl.when(g < B)
        def _():
            pltpu.sync_copy(sh_h.at[pl.ds(g * NPER, NPER)], gbuf)

            def _acc(r, accs):
                out = []
                for c in range(H // 16):
                    sl = pl.ds(c * 16, 16)
                    v = gbuf[r, sl] + b_v[sl]
                    v = jnp.where(v >= 0.0, v, v * alpha)
                    out.append(accs[c] + v)
                return tuple(out)

            accs = lax.fori_loop(
                0, NPER, _acc,
                tuple(jnp.zeros((16,), jnp.float32) for _ in range(H // 16)))
            for c in range(H // 16):
                prow[pl.ds(c * 16, 16)] = accs[c] * (1.0 / NPER)
            pltpu.sync_copy(prow, out_hbm.at[half, g])


_SC_MESH = plsc.VectorSubcoreMesh(
    core_axis_name="c", subcore_axis_name="s", num_cores=NC, num_subcores=NS)

_sc_aggregate = pl.kernel(
    _sc_body,
    out_type=jax.ShapeDtypeStruct((NC, B, H), jnp.float32),
    mesh=_SC_MESH,
    compiler_params=pltpu.CompilerParams(needs_layout_passes=False),
    scratch_types=[
        pltpu.VMEM((G, K), jnp.int32),
        pltpu.VMEM((G, K), jnp.int32),
        pltpu.VMEM((G, K), jnp.float32),
        pltpu.VMEM((K, H), jnp.float32),
        pltpu.VMEM((NPER, H), jnp.float32),
        pltpu.VMEM((H,), jnp.float32),
        pltpu.VMEM((16,), jnp.float32),
        pltpu.VMEM((H,), jnp.float32),
        pltpu.VMEM_SHARED((N, H), jnp.float32),
    ],
)


def _norm_body(parts_ref, aout_ref, pooled_ref, anch_ref):
    p0 = parts_ref[0]
    p1 = parts_ref[1]
    ss = (jnp.sum(p0 * p0, axis=1, keepdims=True)
          + jnp.sum(p1 * p1, axis=1, keepdims=True))
    d = jnp.maximum(jnp.sqrt(ss), 1e-12)
    pooled_ref[:, :H] = p0 / d
    pooled_ref[:, H:] = p1 / d
    a = aout_ref[...]
    da = jnp.maximum(jnp.sqrt(jnp.sum(a * a, axis=1, keepdims=True)), 1e-12)
    anch_ref[...] = a / da


def _phase_c(parts, anchor_out):
    return pl.pallas_call(
        _norm_body,
        out_shape=[
            jax.ShapeDtypeStruct((B, D), jnp.float32),
            jax.ShapeDtypeStruct((B, D), jnp.float32),
        ],
    )(parts, anchor_out)


def kernel(x, edge_index, edge_w, W, b, prelu_a):
    x = x.astype(jnp.float32)
    anchors = x.reshape(B, NPER, D)[:, 0, :]
    b2 = b.astype(jnp.float32).reshape(1, D)
    feat2, anchor_out = _phase_a(x, W.astype(jnp.float32), b2, anchors)

    src = edge_index[0].astype(jnp.int32)
    dst = edge_index[1].astype(jnp.int32)
    pad = EP - E
    zpad_i = jnp.zeros((pad,), jnp.int32)
    srcp = jnp.concatenate([src, zpad_i]).reshape(NS, CH, K)
    dstp = jnp.concatenate([dst, zpad_i]).reshape(NS, CH, K)
    wp = jnp.concatenate(
        [edge_w.astype(jnp.float32), jnp.zeros((pad,), jnp.float32)]
    ).reshape(NS, CH, K)
    bhalf = b.astype(jnp.float32).reshape(NC, H)
    a16 = jnp.broadcast_to(prelu_a.astype(jnp.float32), (16,))
    feat_flat = feat2.reshape(2 * N, H)

    parts = _sc_aggregate(feat_flat, srcp, dstp, wp, bhalf, a16)
    pooled_n, anchor_n = _phase_c(parts, anchor_out)
    return (pooled_n, anchor_n)


# speedup vs baseline: 3.4018x; 1.2642x over previous
import functools

import jax
import jax.numpy as jnp
from jax import lax
from jax.experimental import pallas as pl
from jax.experimental.pallas import tpu as pltpu
from jax.experimental.pallas import tpu_sc as plsc

N = 10000
B = 100
NPER = 100
D = 256
H = 128
E = 160000
NC = 2
NS = 16
K = 128
G = 8
NG = 10
CH = G * NG
EPT = CH * K
EP = NS * EPT
RB = 2000


def _mm_body(x_ref, w_ref, b_ref, anch_ref, feat_ref, aout_ref):
    i = pl.program_id(0)
    xb = x_ref[...]
    row = lax.broadcasted_iota(jnp.int32, (RB, 1), 0) + i * RB
    xb = jnp.where(row % NPER == 0, 0.0, xb)
    f = jnp.dot(xb, w_ref[...], preferred_element_type=jnp.float32)
    feat_ref[0] = f[:, :H]
    feat_ref[1] = f[:, H:]

    @pl.when(i == 0)
    def _():
        aout_ref[...] = (
            jnp.dot(anch_ref[...], w_ref[...], preferred_element_type=jnp.float32)
            + b_ref[...]
        )


def _phase_a(x, W, b2, anchors):
    return pl.pallas_call(
        _mm_body,
        grid=(N // RB,),
        in_specs=[
            pl.BlockSpec((RB, D), lambda i: (i, 0)),
            pl.BlockSpec((D, D), lambda i: (0, 0)),
            pl.BlockSpec((1, D), lambda i: (0, 0)),
            pl.BlockSpec((B, D), lambda i: (0, 0)),
        ],
        out_specs=[
            pl.BlockSpec((2, RB, H), lambda i: (0, i, 0)),
            pl.BlockSpec((B, D), lambda i: (0, 0)),
        ],
        out_shape=[
            jax.ShapeDtypeStruct((2, N, H), jnp.float32),
            jax.ShapeDtypeStruct((B, D), jnp.float32),
        ],
    )(x, W, b2, anchors)


def _sc_body(feat_hbm, src_hbm, dst_hbm, w_hbm, b_hbm, a_hbm, out_hbm,
             eg_src, eg_dst, eg_w, buf0, buf1, b_v, a_v, prow,
             gsem0, gsem1, ssem0, ssem1, sh_h):
    half = lax.axis_index("c")
    t = lax.axis_index("s")
    bufs = (buf0, buf1)
    gsems = (gsem0, gsem1)
    ssems = (ssem0, ssem1)

    def _zrow(r, _):
        for c in range(H // 16):
            buf0[r, pl.ds(c * 16, 16)] = jnp.zeros((16,), jnp.float32)
        return 0

    lax.fori_loop(0, K, _zrow, 0)
    ZR = 125
    for i in range(5):
        pltpu.sync_copy(buf0.at[pl.ds(0, ZR)],
                        sh_h.at[pl.ds(t * 625 + i * ZR, ZR)])

    pltpu.sync_copy(b_hbm.at[half], b_v)
    pltpu.sync_copy(a_hbm, a_v)

    off = half * N

    plsc.subcore_barrier()

    def _gather(j, gi):
        p = j % 2
        return pltpu.make_async_copy(feat_hbm.at[eg_src.at[j]], bufs[p],
                                     gsems[p])

    def _scatter(j):
        p = j % 2
        return pltpu.make_async_copy(bufs[p], sh_h.at[eg_dst.at[j]], ssems[p])

    def _scale(j):
        p = j % 2
        buf = bufs[p]

        def _body(k, _):
            wk = plsc.load_gather(
                eg_w, [jnp.full((16,), j, jnp.int32),
                       jnp.full((16,), k, jnp.int32)])
            for c in range(H // 16):
                sl = pl.ds(c * 16, 16)
                buf[k, sl] = buf[k, sl] * wk
            return 0

        lax.fori_loop(0, K, _body, 0, unroll=2)

    def _group(gi, _):
        gsl = pl.ds(gi * G, G)
        pltpu.sync_copy(src_hbm.at[t, gsl], eg_src)
        pltpu.sync_copy(dst_hbm.at[t, gsl], eg_dst)
        pltpu.sync_copy(w_hbm.at[t, gsl], eg_w)
        for j in range(G):
            for c in range(K // 16):
                sl = pl.ds(c * 16, 16)
                eg_src[j, sl] = eg_src[j, sl] + off

        _gather(0, gi).start()
        for j in range(G):
            if j + 1 < G:
                if j >= 1:
                    _scatter(j - 1).wait()
                _gather(j + 1, gi).start()
            _gather(j, gi).wait()
            _scale(j)
            _scatter(j).start(add=True)
        _scatter(G - 2).wait()
        _scatter(G - 1).wait()
        return 0

    lax.fori_loop(0, NG, _group, 0)

    plsc.subcore_barrier()

    alpha = a_v[...]
    for gi in range(7):
        g = t + NS * gi

        @pl.when(g < B)
        def _():
            pltpu.sync_copy(sh_h.at[pl.ds(g * NPER, NPER)],
                            buf0.at[pl.ds(0, NPER)])

            def _acc(r, accs):
                out = []
                for c in range(H // 16):
                    sl = pl.ds(c * 16, 16)
                    v = buf0[r, sl] + b_v[sl]
                    v = jnp.where(v >= 0.0, v, v * alpha)
                    out.append(accs[c] + v)
                return tuple(out)

            accs = lax.fori_loop(
                0, NPER, _acc,
                tuple(jnp.zeros((16,), jnp.float32) for _ in range(H // 16)))
            for c in range(H // 16):
                prow[pl.ds(c * 16, 16)] = accs[c] * (1.0 / NPER)
            pltpu.sync_copy(prow, out_hbm.at[half, g])


_SC_MESH = plsc.VectorSubcoreMesh(
    core_axis_name="c", subcore_axis_name="s", num_cores=NC, num_subcores=NS)

_sc_aggregate = pl.kernel(
    _sc_body,
    out_type=jax.ShapeDtypeStruct((NC, B, H), jnp.float32),
    mesh=_SC_MESH,
    compiler_params=pltpu.CompilerParams(needs_layout_passes=False),
    scratch_types=[
        pltpu.VMEM((G, K), jnp.int32),
        pltpu.VMEM((G, K), jnp.int32),
        pltpu.VMEM((G, K), jnp.float32),
        pltpu.VMEM((K, H), jnp.float32),
        pltpu.VMEM((K, H), jnp.float32),
        pltpu.VMEM((H,), jnp.float32),
        pltpu.VMEM((16,), jnp.float32),
        pltpu.VMEM((H,), jnp.float32),
        pltpu.SemaphoreType.DMA,
        pltpu.SemaphoreType.DMA,
        pltpu.SemaphoreType.DMA,
        pltpu.SemaphoreType.DMA,
        pltpu.VMEM_SHARED((N, H), jnp.float32),
    ],
)


def _norm_body(parts_ref, aout_ref, pooled_ref, anch_ref):
    p0 = parts_ref[0]
    p1 = parts_ref[1]
    ss = (jnp.sum(p0 * p0, axis=1, keepdims=True)
          + jnp.sum(p1 * p1, axis=1, keepdims=True))
    d = jnp.maximum(jnp.sqrt(ss), 1e-12)
    pooled_ref[:, :H] = p0 / d
    pooled_ref[:, H:] = p1 / d
    a = aout_ref[...]
    da = jnp.maximum(jnp.sqrt(jnp.sum(a * a, axis=1, keepdims=True)), 1e-12)
    anch_ref[...] = a / da


def _phase_c(parts, anchor_out):
    return pl.pallas_call(
        _norm_body,
        out_shape=[
            jax.ShapeDtypeStruct((B, D), jnp.float32),
            jax.ShapeDtypeStruct((B, D), jnp.float32),
        ],
    )(parts, anchor_out)


def kernel(x, edge_index, edge_w, W, b, prelu_a):
    x = x.astype(jnp.float32)
    anchors = x.reshape(B, NPER, D)[:, 0, :]
    b2 = b.astype(jnp.float32).reshape(1, D)
    feat2, anchor_out = _phase_a(x, W.astype(jnp.float32), b2, anchors)

    src = edge_index[0].astype(jnp.int32)
    dst = edge_index[1].astype(jnp.int32)
    pad = EP - E
    zpad_i = jnp.zeros((pad,), jnp.int32)
    srcp = jnp.concatenate([src, zpad_i]).reshape(NS, CH, K)
    dstp = jnp.concatenate([dst, zpad_i]).reshape(NS, CH, K)
    wp = jnp.concatenate(
        [edge_w.astype(jnp.float32), jnp.zeros((pad,), jnp.float32)]
    ).reshape(NS, CH, K)
    bhalf = b.astype(jnp.float32).reshape(NC, H)
    a16 = jnp.broadcast_to(prelu_a.astype(jnp.float32), (16,))
    feat_flat = feat2.reshape(2 * N, H)

    parts = _sc_aggregate(feat_flat, srcp, dstp, wp, bhalf, a16)
    pooled_n, anchor_n = _phase_c(parts, anchor_out)
    return (pooled_n, anchor_n)
